# baseline (speedup 1.0000x reference)
import functools

import jax
import jax.numpy as jnp
from jax import lax
from jax.experimental import pallas as pl
from jax.experimental.pallas import tpu as pltpu
from jax.experimental.pallas import tpu_sc as plsc

_VOCAB = 100000
_D = 128
_BATCH = 4096
_HIST = 50
_NW = 32
_EPW = _BATCH // _NW
_HPC = 1
_NB = 7
_NCHUNK = _HIST // _HPC


def _make_sc_gather():
    mesh = plsc.VectorSubcoreMesh(core_axis_name="c", subcore_axis_name="s")

    @functools.partial(
        pl.kernel,
        mesh=mesh,
        out_type=jax.ShapeDtypeStruct((_HIST, _BATCH, _D), jnp.float32),
        scratch_types=[
            pltpu.VMEM((_HIST, _EPW), jnp.int32),
        ] + [pltpu.VMEM((_HPC, _EPW, _D), jnp.float32)] * _NB
          + [pltpu.SemaphoreType.DMA] * (2 * _NB),
    )
    def sc_gather(idx_hbm, table_hbm, out_hbm, idx_v, *rest):
        bufs = list(rest[:_NB])
        gsem = list(rest[_NB:2 * _NB])
        ssem = list(rest[2 * _NB:3 * _NB])
        wid = lax.axis_index("s") * 2 + lax.axis_index("c")
        ebase = wid * _EPW
        pltpu.sync_copy(idx_hbm.at[:, pl.ds(ebase, _EPW)], idx_v)

        def gather(g, b):
            for j in range(_HPC):
                pltpu.async_copy(
                    table_hbm.at[idx_v.at[g * _HPC + j]],
                    bufs[b].at[j], gsem[b])

        def gwait(g, b):
            for j in range(_HPC):
                pltpu.make_async_copy(
                    table_hbm.at[idx_v.at[g * _HPC + j]],
                    bufs[b].at[j], gsem[b]).wait()

        def astore(g, b):
            pltpu.async_copy(
                bufs[b],
                out_hbm.at[pl.ds(g * _HPC, _HPC), pl.ds(ebase, _EPW)],
                ssem[b])

        def swait(g, b):
            pltpu.make_async_copy(
                bufs[b],
                out_hbm.at[pl.ds(g * _HPC, _HPC), pl.ds(ebase, _EPW)],
                ssem[b]).wait()

        for b in range(_NB - 1):
            gather(b, b)
        gwait(0, 0)
        astore(0, 0)
        gather(_NB - 1, _NB - 1)

        def body(t, carry):
            gbase = _NB * t + 1
            for k in range(_NB):
                g = gbase + k
                b = (1 + k) % _NB
                gwait(g, b)
                swait(g - 1, (b - 1) % _NB)
                gather(g + _NB - 1, (b - 1) % _NB)
                astore(g, b)
            return carry

        T = (_NCHUNK - _NB) // _NB
        lax.fori_loop(0, T, body, 0)

        waited = _NB * T
        for g in range(_NB * T + 1, _NCHUNK):
            gwait(g, g % _NB)
            nxt = g + _NB - 1
            if nxt <= _NCHUNK - 1:
                swait(g - 1, (g - 1) % _NB)
                waited = g
                gather(nxt, nxt % _NB)
            astore(g, g % _NB)
        for g in range(waited, _NCHUNK):
            swait(g, g % _NB)

    return sc_gather


_sc_gather = _make_sc_gather()


def kernel(x, table):
    xt = x.T.astype(jnp.int32)
    out = _sc_gather(xt, table)
    return jnp.transpose(out, (1, 0, 2))
